# baseline (speedup 1.0000x reference)
import functools

import jax
import jax.numpy as jnp
from jax import lax
from jax.experimental import pallas as pl
from jax.experimental.pallas import tpu as pltpu
from jax.experimental.pallas import tpu_sc as plsc

_N = 12800
_TL = 8
_TE = 128
_PL = 16
_AH = 128
_NW = 32
_CTX = 2 * _N
_CPW = _CTX // _NW
_CCH = 16
_NCH = _CPW // _CCH


def _tok_gather_sum(idx_flat, tok_emb):
    mesh = plsc.VectorSubcoreMesh(core_axis_name="c", subcore_axis_name="s")

    @functools.partial(
        pl.kernel,
        mesh=mesh,
        out_type=jax.ShapeDtypeStruct((_CTX, _TE), jnp.float32),
        scratch_types=[
            pltpu.VMEM((_CCH * _TL,), jnp.int32),
            pltpu.VMEM((_CCH * _TL, _TE), jnp.float32),
            pltpu.VMEM((_CCH, _TE), jnp.float32),
            pltpu.SemaphoreType.DMA,
        ],
    )
    def k(idx_hbm, table_hbm, out_hbm, idx_v, rows_v, acc_v, sem):
        wid = lax.axis_index("s") * 2 + lax.axis_index("c")
        ctx0 = wid * _CPW

        def chunk(kk, carry):
            cbase = ctx0 + kk * _CCH
            pltpu.sync_copy(idx_hbm.at[pl.ds(cbase * _TL, _CCH * _TL)], idx_v)
            pltpu.async_copy(table_hbm.at[idx_v], rows_v, sem).wait()

            def ctx_body(ci, c2):
                r0 = ci * _TL
                for v in range(_TE // 16):
                    s = rows_v[r0, pl.ds(v * 16, 16)]
                    for j in range(1, _TL):
                        s = s + rows_v[r0 + j, pl.ds(v * 16, 16)]
                    acc_v[ci, pl.ds(v * 16, 16)] = s
                return c2

            lax.fori_loop(0, _CCH, ctx_body, 0)
            pltpu.sync_copy(acc_v, out_hbm.at[pl.ds(cbase, _CCH)])
            return carry

        lax.fori_loop(0, _NCH, chunk, 0)

    return k(idx_flat, tok_emb)


_R = 512


def _lstm_body(path_ref, lens_ref, ast_ref, wih_ref, wfhh_ref, bf_ref,
               wbhh_ref, bb_ref, out_ref, px_ref, xw_ref):
    pth = path_ref[...]
    lm1 = jnp.maximum(lens_ref[...], 1) - 1
    ast = ast_ref[...]
    wcat = wih_ref[...]
    av_iota = lax.broadcasted_iota(jnp.int32, (_R, 512), 1)

    for t in range(_PL):
        oh = (pth[:, t][:, None] == av_iota).astype(jnp.bfloat16)
        px_ref[pl.ds(t * _R, _R), :] = jnp.dot(
            oh, ast, preferred_element_type=jnp.float32
        ).astype(jnp.bfloat16)
    xw_ref[...] = lax.dot_general(
        px_ref[...], wcat, (((1,), (1,)), ((), ())),
        preferred_element_type=jnp.float32).astype(jnp.bfloat16)

    wfhh = wfhh_ref[...]
    wbhh = wbhh_ref[...]
    bf = bf_ref[...]
    bb = bb_ref[...]
    zeros = jnp.zeros((_R, _AH), jnp.float32)
    hf, cf, hb, cb, hfc = zeros, zeros, zeros, zeros, zeros

    def gates(x, h, whh, b):
        g = x.astype(jnp.float32) + lax.dot_general(
            h.astype(jnp.bfloat16), whh, (((1,), (1,)), ((), ())),
            preferred_element_type=jnp.float32) + b
        i_ = 0.5 * jnp.tanh(g[:, 0 * _AH:1 * _AH]) + 0.5
        f_ = 0.5 * jnp.tanh(g[:, 1 * _AH:2 * _AH]) + 0.5
        g_ = jnp.tanh(g[:, 2 * _AH:3 * _AH])
        o_ = 0.5 * jnp.tanh(g[:, 3 * _AH:4 * _AH]) + 0.5
        return i_, f_, g_, o_

    for s in range(_PL):
        tb = _PL - 1 - s
        i_, f_, g_, o_ = gates(xw_ref[pl.ds(s * _R, _R), :4 * _AH], hf, wfhh, bf)
        cf = f_ * cf + i_ * g_
        hf = o_ * jnp.tanh(cf)
        hfc = jnp.where(lm1 == s, hf, hfc)
        rm = (lm1 == tb)
        hbi = jnp.where(rm, 0.0, hb)
        cbi = jnp.where(rm, 0.0, cb)
        ib, fb, gb, ob = gates(xw_ref[pl.ds(tb * _R, _R), 4 * _AH:], hbi, wbhh, bb)
        cb = fb * cbi + ib * gb
        hb = ob * jnp.tanh(cb)

    out_ref[:, :_AH] = hfc
    out_ref[:, _AH:] = hb


def _lstm_call(path2, lens2, ast_emb, wih, wfhh, bf, wbhh, bb):
    av = ast_emb.shape[0]
    return pl.pallas_call(
        _lstm_body,
        grid=(_N // _R,),
        in_specs=[
            pl.BlockSpec((_R, _PL), lambda i: (i, 0)),
            pl.BlockSpec((_R, 1), lambda i: (i, 0)),
            pl.BlockSpec((av, _TE), lambda i: (0, 0)),
            pl.BlockSpec((8 * _AH, _TE), lambda i: (0, 0)),
            pl.BlockSpec((4 * _AH, _AH), lambda i: (0, 0)),
            pl.BlockSpec((1, 4 * _AH), lambda i: (0, 0)),
            pl.BlockSpec((4 * _AH, _AH), lambda i: (0, 0)),
            pl.BlockSpec((1, 4 * _AH), lambda i: (0, 0)),
        ],
        out_specs=pl.BlockSpec((_R, 2 * _AH), lambda i: (i, 0)),
        out_shape=jax.ShapeDtypeStruct((_N, 2 * _AH), jnp.float32),
        scratch_shapes=[pltpu.VMEM((_PL * _R, _TE), jnp.bfloat16),
                        pltpu.VMEM((_PL * _R, 8 * _AH), jnp.bfloat16)],
    )(path2, lens2, ast_emb, wih, wfhh, bf, wbhh, bb)


_R3 = 256


def _fc_body(st_ref, pe_ref, en_ref, lens_ref, w_ref, b_ref, out_ref):
    m = (lens_ref[...] > 0).astype(jnp.float32)
    comb = jnp.concatenate(
        [st_ref[...], pe_ref[...], en_ref[...]], axis=1) * m
    out_ref[...] = jnp.tanh(
        lax.dot_general(comb, w_ref[...], (((1,), (1,)), ((), ())),
                        preferred_element_type=jnp.float32) + b_ref[...])


def _fc_call(st_e, pe, en_e, lens2, fc_w, fc_b):
    ch, fin = fc_w.shape
    return pl.pallas_call(
        _fc_body,
        grid=(_N // _R3,),
        in_specs=[
            pl.BlockSpec((_R3, _TE), lambda i: (i, 0)),
            pl.BlockSpec((_R3, 2 * _AH), lambda i: (i, 0)),
            pl.BlockSpec((_R3, _TE), lambda i: (i, 0)),
            pl.BlockSpec((_R3, 1), lambda i: (i, 0)),
            pl.BlockSpec((ch, fin), lambda i: (0, 0)),
            pl.BlockSpec((1, ch), lambda i: (0, 0)),
        ],
        out_specs=pl.BlockSpec((_R3, ch), lambda i: (i, 0)),
        out_shape=jax.ShapeDtypeStruct((_N, ch), jnp.float32),
    )(st_e, pe, en_e, lens2, fc_w, fc_b)


def kernel(start, end, path, start_len, end_len, ast_path_lens,
           tok_emb, ast_emb, Wf_ih, Wf_hh, bf, Wb_ih, Wb_hh, bb, fc_W, fc_b):
    bsz, mc = start.shape[0], start.shape[1]
    n = bsz * mc
    st = start.reshape(n, -1).astype(jnp.int32)
    en = end.reshape(n, -1).astype(jnp.int32)
    idx = jnp.concatenate([st, en], axis=0).reshape(-1)
    gath = _tok_gather_sum(idx, tok_emb)
    st_e, en_e = gath[:n], gath[n:]

    path2 = path.reshape(n, -1).astype(jnp.int32)
    lens2 = ast_path_lens.reshape(n, 1).astype(jnp.int32)
    gs = jnp.concatenate([jnp.full((2 * _AH,), 0.5, jnp.float32),
                          jnp.ones((_AH,), jnp.float32),
                          jnp.full((_AH,), 0.5, jnp.float32)])[:, None]
    wih = jnp.concatenate([Wf_ih * gs, Wb_ih * gs], axis=0)
    pe = _lstm_call(path2, lens2, ast_emb.astype(jnp.bfloat16),
                    wih.astype(jnp.bfloat16),
                    (Wf_hh * gs).astype(jnp.bfloat16),
                    (bf * gs[:, 0]).reshape(1, -1),
                    (Wb_hh * gs).astype(jnp.bfloat16),
                    (bb * gs[:, 0]).reshape(1, -1))

    out = _fc_call(st_e, pe, en_e, lens2, fc_W, fc_b.reshape(1, -1))
    return out.reshape(bsz, mc, fc_W.shape[0])


# speedup vs baseline: 1.0660x; 1.0660x over previous
import functools

import jax
import jax.numpy as jnp
from jax import lax
from jax.experimental import pallas as pl
from jax.experimental.pallas import tpu as pltpu
from jax.experimental.pallas import tpu_sc as plsc

_N = 12800
_TL = 8
_TE = 128
_PL = 16
_AH = 128
_NW = 32
_CTX = 2 * _N
_CPW = _CTX // _NW
_CCH = 16
_NCH = _CPW // _CCH


def _tok_gather_sum(idx_flat, tok_emb):
    mesh = plsc.VectorSubcoreMesh(core_axis_name="c", subcore_axis_name="s")

    @functools.partial(
        pl.kernel,
        mesh=mesh,
        out_type=jax.ShapeDtypeStruct((_CTX, _TE), jnp.float32),
        scratch_types=[
            pltpu.VMEM((_CCH * _TL,), jnp.int32),
            pltpu.VMEM((_CCH * _TL, _TE), jnp.float32),
            pltpu.VMEM((_CCH, _TE), jnp.float32),
            pltpu.SemaphoreType.DMA,
        ],
    )
    def k(idx_hbm, table_hbm, out_hbm, idx_v, rows_v, acc_v, sem):
        wid = lax.axis_index("s") * 2 + lax.axis_index("c")
        ctx0 = wid * _CPW

        def chunk(kk, carry):
            cbase = ctx0 + kk * _CCH
            pltpu.sync_copy(idx_hbm.at[pl.ds(cbase * _TL, _CCH * _TL)], idx_v)
            pltpu.async_copy(table_hbm.at[idx_v], rows_v, sem).wait()

            def ctx_body(ci, c2):
                r0 = ci * _TL
                for v in range(_TE // 16):
                    s = rows_v[r0, pl.ds(v * 16, 16)]
                    for j in range(1, _TL):
                        s = s + rows_v[r0 + j, pl.ds(v * 16, 16)]
                    acc_v[ci, pl.ds(v * 16, 16)] = s
                return c2

            lax.fori_loop(0, _CCH, ctx_body, 0)
            pltpu.sync_copy(acc_v, out_hbm.at[pl.ds(cbase, _CCH)])
            return carry

        lax.fori_loop(0, _NCH, chunk, 0)

    return k(idx_flat, tok_emb)


_R = 640


def _lstm_body(path_ref, lens_ref, ast_ref, wih_ref, wfhh_ref, bf_ref,
               wbhh_ref, bb_ref, out_ref, px_ref, xw_ref):
    pth = path_ref[...]
    lm1 = jnp.maximum(lens_ref[...], 1) - 1
    ast = ast_ref[...]
    wcat = wih_ref[...]
    av_iota = lax.broadcasted_iota(jnp.int32, (_R, 512), 1)

    for t in range(_PL):
        oh = (pth[:, t][:, None] == av_iota).astype(jnp.bfloat16)
        px_ref[pl.ds(t * _R, _R), :] = jnp.dot(
            oh, ast, preferred_element_type=jnp.float32
        ).astype(jnp.bfloat16)
    xw_ref[...] = lax.dot_general(
        px_ref[...], wcat, (((1,), (1,)), ((), ())),
        preferred_element_type=jnp.float32)

    wfhh = wfhh_ref[...]
    wbhh = wbhh_ref[...]
    bf = bf_ref[...]
    bb = bb_ref[...]
    zeros = jnp.zeros((_R, _AH), jnp.float32)
    hf, cf, hb, cb, hfc = zeros, zeros, zeros, zeros, zeros

    def gates(x, h, whh, b):
        g = x + lax.dot_general(
            h.astype(jnp.bfloat16), whh, (((1,), (1,)), ((), ())),
            preferred_element_type=jnp.float32) + b
        i_ = 0.5 * jnp.tanh(g[:, 0 * _AH:1 * _AH]) + 0.5
        f_ = 0.5 * jnp.tanh(g[:, 1 * _AH:2 * _AH]) + 0.5
        g_ = jnp.tanh(g[:, 2 * _AH:3 * _AH])
        o_ = 0.5 * jnp.tanh(g[:, 3 * _AH:4 * _AH]) + 0.5
        return i_, f_, g_, o_

    for s in range(_PL):
        tb = _PL - 1 - s
        i_, f_, g_, o_ = gates(xw_ref[pl.ds(s * _R, _R), :4 * _AH], hf, wfhh, bf)
        cf = f_ * cf + i_ * g_
        hf = o_ * jnp.tanh(cf)
        hfc = jnp.where(lm1 == s, hf, hfc)
        rm = (lm1 == tb)
        hbi = jnp.where(rm, 0.0, hb)
        cbi = jnp.where(rm, 0.0, cb)
        ib, fb, gb, ob = gates(xw_ref[pl.ds(tb * _R, _R), 4 * _AH:], hbi, wbhh, bb)
        cb = fb * cbi + ib * gb
        hb = ob * jnp.tanh(cb)

    out_ref[:, :_AH] = hfc
    out_ref[:, _AH:] = hb


def _lstm_call(path2, lens2, ast_emb, wih, wfhh, bf, wbhh, bb):
    av = ast_emb.shape[0]
    return pl.pallas_call(
        _lstm_body,
        grid=(_N // _R,),
        in_specs=[
            pl.BlockSpec((_R, _PL), lambda i: (i, 0)),
            pl.BlockSpec((_R, 1), lambda i: (i, 0)),
            pl.BlockSpec((av, _TE), lambda i: (0, 0)),
            pl.BlockSpec((8 * _AH, _TE), lambda i: (0, 0)),
            pl.BlockSpec((4 * _AH, _AH), lambda i: (0, 0)),
            pl.BlockSpec((1, 4 * _AH), lambda i: (0, 0)),
            pl.BlockSpec((4 * _AH, _AH), lambda i: (0, 0)),
            pl.BlockSpec((1, 4 * _AH), lambda i: (0, 0)),
        ],
        out_specs=pl.BlockSpec((_R, 2 * _AH), lambda i: (i, 0)),
        out_shape=jax.ShapeDtypeStruct((_N, 2 * _AH), jnp.float32),
        scratch_shapes=[pltpu.VMEM((_PL * _R, _TE), jnp.bfloat16),
                        pltpu.VMEM((_PL * _R, 8 * _AH), jnp.float32)],
    )(path2, lens2, ast_emb, wih, wfhh, bf, wbhh, bb)


_R3 = 256


def _fc_body(st_ref, pe_ref, en_ref, lens_ref, w_ref, b_ref, out_ref):
    m = (lens_ref[...] > 0).astype(jnp.float32)
    comb = jnp.concatenate(
        [st_ref[...], pe_ref[...], en_ref[...]], axis=1) * m
    out_ref[...] = jnp.tanh(
        lax.dot_general(comb, w_ref[...], (((1,), (1,)), ((), ())),
                        preferred_element_type=jnp.float32) + b_ref[...])


def _fc_call(st_e, pe, en_e, lens2, fc_w, fc_b):
    ch, fin = fc_w.shape
    return pl.pallas_call(
        _fc_body,
        grid=(_N // _R3,),
        in_specs=[
            pl.BlockSpec((_R3, _TE), lambda i: (i, 0)),
            pl.BlockSpec((_R3, 2 * _AH), lambda i: (i, 0)),
            pl.BlockSpec((_R3, _TE), lambda i: (i, 0)),
            pl.BlockSpec((_R3, 1), lambda i: (i, 0)),
            pl.BlockSpec((ch, fin), lambda i: (0, 0)),
            pl.BlockSpec((1, ch), lambda i: (0, 0)),
        ],
        out_specs=pl.BlockSpec((_R3, ch), lambda i: (i, 0)),
        out_shape=jax.ShapeDtypeStruct((_N, ch), jnp.float32),
    )(st_e, pe, en_e, lens2, fc_w, fc_b)


def kernel(start, end, path, start_len, end_len, ast_path_lens,
           tok_emb, ast_emb, Wf_ih, Wf_hh, bf, Wb_ih, Wb_hh, bb, fc_W, fc_b):
    bsz, mc = start.shape[0], start.shape[1]
    n = bsz * mc
    st = start.reshape(n, -1).astype(jnp.int32)
    en = end.reshape(n, -1).astype(jnp.int32)
    idx = jnp.concatenate([st, en], axis=0).reshape(-1)
    gath = _tok_gather_sum(idx, tok_emb)
    st_e, en_e = gath[:n], gath[n:]

    path2 = path.reshape(n, -1).astype(jnp.int32)
    lens2 = ast_path_lens.reshape(n, 1).astype(jnp.int32)
    gs = jnp.concatenate([jnp.full((2 * _AH,), 0.5, jnp.float32),
                          jnp.ones((_AH,), jnp.float32),
                          jnp.full((_AH,), 0.5, jnp.float32)])[:, None]
    wih = jnp.concatenate([Wf_ih * gs, Wb_ih * gs], axis=0)
    pe = _lstm_call(path2, lens2, ast_emb.astype(jnp.bfloat16),
                    wih.astype(jnp.bfloat16),
                    (Wf_hh * gs).astype(jnp.bfloat16),
                    (bf * gs[:, 0]).reshape(1, -1),
                    (Wb_hh * gs).astype(jnp.bfloat16),
                    (bb * gs[:, 0]).reshape(1, -1))

    out = _fc_call(st_e, pe, en_e, lens2, fc_W, fc_b.reshape(1, -1))
    return out.reshape(bsz, mc, fc_W.shape[0])
